# baseline (speedup 1.0000x reference)
import functools

import jax
import jax.numpy as jnp
from jax import lax
from jax.experimental import pallas as pl
from jax.experimental.pallas import tpu as pltpu
from jax.experimental.pallas import tpu_sc as plsc

_ROWS = 128
_VOCAB = 100000
_SAVE = 64
_OUT_W = 72
_LANES = 16
_NVEC = _VOCAB // _LANES
_BIG = 2**30


def _body(logits_hbm, save_id_hbm, out_hbm, rowbuf, outbuf):
    nc, ns = 2, 16
    wid = lax.axis_index("s") * nc + lax.axis_index("c")
    rows_per_w = _ROWS // (nc * ns)
    lane_iota = lax.iota(jnp.int32, _LANES)

    for k in range(rows_per_w):
        row = wid * rows_per_w + k
        pltpu.sync_copy(logits_hbm.at[pl.ds(row * _VOCAB, _VOCAB)], rowbuf)
        pltpu.sync_copy(save_id_hbm.at[pl.ds(row * _SAVE, _SAVE)],
                        outbuf.at[pl.ds(0, _SAVE)])

        def step(i, carry):
            vmax, vidx = carry
            x = rowbuf[pl.ds(i * _LANES, _LANES)]
            upd = x > vmax
            vmax = jnp.where(upd, x, vmax)
            vidx = jnp.where(upd, i * _LANES + lane_iota, vidx)
            return vmax, vidx

        init = (jnp.full((_LANES,), -jnp.inf, jnp.float32),
                jnp.zeros((_LANES,), jnp.int32))
        vmax, vidx = lax.fori_loop(0, _NVEC, step, init)

        bv = vmax[0]
        bi = vidx[0]
        for j in range(1, _LANES):
            v = vmax[j]
            iv = vidx[j]
            better = (v > bv) | ((v == bv) & (iv < bi))
            bv = jnp.where(better, v, bv)
            bi = jnp.where(better, iv, bi)
        gidx = bi

        outbuf[pl.ds(_SAVE, _LANES)] = jnp.full((_LANES,), gidx, jnp.int32)
        pltpu.sync_copy(outbuf.at[pl.ds(0, _OUT_W)],
                        out_hbm.at[pl.ds(row * _OUT_W, _OUT_W)])


@jax.jit
def _greedy(logits, save_id):
    mesh = plsc.VectorSubcoreMesh(core_axis_name="c", subcore_axis_name="s")
    f = pl.kernel(
        _body,
        out_type=jax.ShapeDtypeStruct((_ROWS * _OUT_W,), jnp.int32),
        mesh=mesh,
        scratch_types=[
            pltpu.VMEM((_VOCAB,), jnp.float32),
            pltpu.VMEM((_SAVE + _LANES,), jnp.int32),
        ],
    )
    out = f(logits.reshape(-1), save_id.reshape(-1))
    return out.reshape(_ROWS, _OUT_W)


def kernel(logits, save_id):
    out = _greedy(logits, save_id)
    return (out[:, _SAVE:_SAVE + 1], out[:, :_SAVE + 1])
